# baseline (speedup 1.0000x reference)
import jax
import jax.numpy as jnp
from jax.experimental import pallas as pl
from jax.experimental.pallas import tpu as pltpu

B, T, N, F, GF = 512, 12, 200, 16, 1
GH, OUT, FCH, NL = 64, 512, 512, 3
EPS = 1e-5


def _bn_cols(h, g, be):
    m = jnp.mean(h, axis=0, keepdims=True)
    v = jnp.mean((h - m) * (h - m), axis=0, keepdims=True)
    return g * (h - m) * jax.lax.rsqrt(v + EPS) + be


def _stgcn_body(x_ref, cheb_ref, wg_ref, bg_ref,
                wx_ref, wh_ref, bl_ref, wout_ref, bout_ref,
                w0_ref, b0_ref, g0_ref, be0_ref,
                g1_ref, be1_ref, w1_ref, b1_ref,
                o_ref, seq, zx):
    sid = x_ref[:, T * N + F - 1:T * N + F].astype(jnp.int32)
    ids = jax.lax.broadcasted_iota(jnp.int32, (B, N), 1)
    onehot = (ids == sid).astype(jnp.float32)
    rows1 = jnp.dot(onehot, cheb_ref[1],
                    preferred_element_type=jnp.float32)

    wg0 = wg_ref[0]
    wg1 = wg_ref[1]
    bg = bg_ref[...]
    for t in range(T):
        xt = x_ref[:, t * N:(t + 1) * N]
        s0 = jnp.sum(onehot * xt, axis=1, keepdims=True)
        s1 = jnp.sum(rows1 * xt, axis=1, keepdims=True)
        xg = jnp.maximum(s0 * wg0 + s1 * wg1 + bg, 0.0)
        seq[t * B:(t + 1) * B, :] = xg

    h = jnp.zeros((B, GH), jnp.float32)
    for l in range(NL):
        wh = wh_ref[l]
        zx[...] = jnp.dot(seq[...], wx_ref[l],
                          preferred_element_type=jnp.float32) + bl_ref[l:l + 1, :]
        h = jnp.zeros((B, GH), jnp.float32)
        c = jnp.zeros((B, GH), jnp.float32)
        for t in range(T):
            z = zx[t * B:(t + 1) * B, :] + jnp.dot(
                h, wh, preferred_element_type=jnp.float32)
            zi = z[:, :GH]
            zf = z[:, GH:2 * GH]
            zg = z[:, 2 * GH:3 * GH]
            zo = z[:, 3 * GH:]
            c = jax.nn.sigmoid(zf) * c + jax.nn.sigmoid(zi) * jnp.tanh(zg)
            h = jax.nn.sigmoid(zo) * jnp.tanh(c)
            seq[t * B:(t + 1) * B, :] = h

    gF = jnp.dot(h, wout_ref[...],
                 preferred_element_type=jnp.float32) + bout_ref[...]

    h2p = jnp.dot(x_ref[:, T * N:T * N + F - 1], w0_ref[...],
                  preferred_element_type=jnp.float32) + b0_ref[...]
    h2n = _bn_cols(h2p, g0_ref[...], be0_ref[...])
    h2 = jnp.where(h2n >= 0, h2n, 0.01 * h2n)
    bna = _bn_cols(h2, g1_ref[:, :FCH], be1_ref[:, :FCH])
    bnb = _bn_cols(gF, g1_ref[:, FCH:], be1_ref[:, FCH:])
    y = (jnp.sum(bna * w1_ref[:, :FCH], axis=1, keepdims=True)
         + jnp.sum(bnb * w1_ref[:, FCH:], axis=1, keepdims=True)
         + b1_ref[...])
    o_ref[...] = jax.nn.sigmoid(y)


_CALL = pl.pallas_call(
    _stgcn_body,
    out_shape=jax.ShapeDtypeStruct((B, 1), jnp.float32),
    scratch_shapes=[
        pltpu.VMEM((T * B, GH), jnp.float32),
        pltpu.VMEM((T * B, 4 * GH), jnp.float32),
    ],
    compiler_params=pltpu.CompilerParams(
        vmem_limit_bytes=100 * 1024 * 1024,
    ),
)


def kernel(x, cheb, W_g, b_g, Wx, Wh, b_lstm, W_out, b_out, W0, b0, g0, be0,
           g1, be1, W1, b1):
    return _CALL(x, cheb, W_g, b_g.reshape(1, GH), Wx, Wh, b_lstm,
                 W_out, b_out.reshape(1, OUT),
                 W0, b0.reshape(1, FCH), g0.reshape(1, FCH),
                 be0.reshape(1, FCH),
                 g1.reshape(1, FCH + OUT), be1.reshape(1, FCH + OUT),
                 W1.reshape(1, FCH + OUT), b1.reshape(1, 1))


# speedup vs baseline: 1.2581x; 1.2581x over previous
import jax
import jax.numpy as jnp
from jax.experimental import pallas as pl
from jax.experimental.pallas import tpu as pltpu

B, T, N, F, GF = 512, 12, 200, 16, 1
GH, OUT, FCH, NL = 64, 512, 512, 3
EPS = 1e-5


def _sig(x):
    return 0.5 * jnp.tanh(0.5 * x) + 0.5


def _bn_cols(h, g, be):
    m = jnp.mean(h, axis=0, keepdims=True)
    v = jnp.mean((h - m) * (h - m), axis=0, keepdims=True)
    return g * (h - m) * jax.lax.rsqrt(v + EPS) + be


def _stgcn_body(x1t_ref, x2_ref, cheb1_ref, wg_ref, bg_ref,
                wx_ref, wh_ref, bl_ref, wout_ref, bout_ref,
                w0_ref, b0_ref, g0_ref, be0_ref,
                g1_ref, be1_ref, w1_ref, b1_ref,
                o_ref, seq, zx):
    sid = x2_ref[:, F - 1:F].astype(jnp.int32)
    ids = jax.lax.broadcasted_iota(jnp.int32, (B, N), 1)
    onehot = (ids == sid).astype(jnp.float32)
    rows1 = jnp.dot(onehot, cheb1_ref[...],
                    preferred_element_type=jnp.float32)

    wg0 = wg_ref[0]
    wg1 = wg_ref[1]
    bg = bg_ref[...]
    for t in range(T):
        xt = x1t_ref[t]
        s0 = jnp.sum(onehot * xt, axis=1, keepdims=True)
        s1 = jnp.sum(rows1 * xt, axis=1, keepdims=True)
        xg = jnp.maximum(s0 * wg0 + s1 * wg1 + bg, 0.0)
        seq[t * B:(t + 1) * B, :] = xg

    h = jnp.zeros((B, GH), jnp.float32)
    for l in range(NL):
        wh = wh_ref[l]
        zx[...] = jnp.dot(seq[...], wx_ref[l],
                          preferred_element_type=jnp.float32) + bl_ref[l:l + 1, :]
        h = jnp.zeros((B, GH), jnp.float32)
        c = jnp.zeros((B, GH), jnp.float32)
        for t in range(T):
            z = zx[t * B:(t + 1) * B, :] + jnp.dot(
                h, wh, preferred_element_type=jnp.float32)
            zi = z[:, :GH]
            zf = z[:, GH:2 * GH]
            zg = z[:, 2 * GH:3 * GH]
            zo = z[:, 3 * GH:]
            c = _sig(zf) * c + _sig(zi) * jnp.tanh(zg)
            h = _sig(zo) * jnp.tanh(c)
            seq[t * B:(t + 1) * B, :] = h

    gF = jnp.dot(h, wout_ref[...],
                 preferred_element_type=jnp.float32) + bout_ref[...]

    h2p = jnp.dot(x2_ref[:, :F - 1], w0_ref[...],
                  preferred_element_type=jnp.float32) + b0_ref[...]
    h2n = _bn_cols(h2p, g0_ref[...], be0_ref[...])
    h2 = jnp.where(h2n >= 0, h2n, 0.01 * h2n)
    bna = _bn_cols(h2, g1_ref[:, :FCH], be1_ref[:, :FCH])
    bnb = _bn_cols(gF, g1_ref[:, FCH:], be1_ref[:, FCH:])
    y = (jnp.sum(bna * w1_ref[:, :FCH], axis=1, keepdims=True)
         + jnp.sum(bnb * w1_ref[:, FCH:], axis=1, keepdims=True)
         + b1_ref[...])
    o_ref[...] = _sig(y)


_CALL = pl.pallas_call(
    _stgcn_body,
    out_shape=jax.ShapeDtypeStruct((B, 1), jnp.float32),
    scratch_shapes=[
        pltpu.VMEM((T * B, GH), jnp.float32),
        pltpu.VMEM((T * B, 4 * GH), jnp.float32),
    ],
    compiler_params=pltpu.CompilerParams(
        vmem_limit_bytes=100 * 1024 * 1024,
    ),
)


def kernel(x, cheb, W_g, b_g, Wx, Wh, b_lstm, W_out, b_out, W0, b0, g0, be0,
           g1, be1, W1, b1):
    x1t = x[:, :T * N * GF].reshape(B, T, N).transpose(1, 0, 2)
    x2 = x[:, T * N * GF:]
    return _CALL(x1t, x2, cheb[1], W_g, b_g.reshape(1, GH), Wx, Wh, b_lstm,
                 W_out, b_out.reshape(1, OUT),
                 W0, b0.reshape(1, FCH), g0.reshape(1, FCH),
                 be0.reshape(1, FCH),
                 g1.reshape(1, FCH + OUT), be1.reshape(1, FCH + OUT),
                 W1.reshape(1, FCH + OUT), b1.reshape(1, 1))
